# baseline (speedup 1.0000x reference)
import jax
import jax.numpy as jnp
from jax import lax
from jax.experimental import pallas as pl
from jax.experimental.pallas import tpu as pltpu
from jax.experimental.pallas import tpu_sc as plsc

_T = 8192
_E = 256
_G = 8
_EPG = 32
_K = 8
_KG = 4
_SCALE = 2.5

_NW = 32
_TPW = _T // _NW
_PAIRS = _TPW // 2
_WORDS = _TPW * _E

_TC_BLK = 1024


def _tc_body(x_ref, b_ref, o_ref):
  x = x_ref[...]
  s4c = jax.nn.sigmoid(x) + b_ref[...]
  o_ref[...] = s4c.reshape(_TC_BLK * _E // 128, 128)


def _rcp(d):
  r = plsc.bitcast(jnp.int32(0x7EF311C7) - plsc.bitcast(d, jnp.int32),
                   jnp.float32)
  for _ in range(3):
    r = r * (jnp.float32(2.0) - d * r)
  return r


def _merge_vals16(ak, bk, dummy):
  hi = jnp.maximum(ak, lax.rev(bk, (0,)))
  sk, _ = plsc.sort_key_val(hi, dummy, descending=True)
  return sk


def _sc_body(s4c_hbm, bias_hbm, w_hbm, id_hbm, buf, bias_v, w_out, id_out,
             stage_k, stage_v):
  c = lax.axis_index("c")
  s = lax.axis_index("s")
  wid = s * 2 + c

  pltpu.sync_copy(bias_hbm, bias_v)
  pltpu.sync_copy(s4c_hbm.at[pl.ds(wid * _WORDS, _WORDS)], buf)

  iota = lax.iota(jnp.int32, 16)
  lane_lt8 = iota < 8
  trans_base = jnp.where(lane_lt8, iota * _EPG,
                         _E + (15 - iota) * _EPG)
  neg_inf = jnp.float32(-jnp.inf)

  @plsc.parallel_loop(0, _PAIRS, unroll=2)
  def pair_body(pair):
    base = pl.multiple_of(pair * (2 * _E), 2 * _E)

    gidx0 = trans_base + base
    m1s, m2s = [], []
    for chunk in range(4):
      m1 = jnp.full((16,), neg_inf, jnp.float32)
      m2 = jnp.full((16,), neg_inf, jnp.float32)
      for j in range(chunk * 8, chunk * 8 + 8):
        x = plsc.load_gather(buf, [gidx0 + j])
        m2 = jnp.maximum(m2, jnp.minimum(m1, x))
        m1 = jnp.maximum(m1, x)
      m1s.append(m1)
      m2s.append(m2)

    def comb(a1, a2, b1, b2):
      return (jnp.maximum(a1, b1),
              jnp.maximum(jnp.minimum(a1, b1), jnp.maximum(a2, b2)))

    x1, x2 = comb(m1s[0], m2s[0], m1s[1], m2s[1])
    y1, y2 = comb(m1s[2], m2s[2], m1s[3], m2s[3])
    g1, g2 = comb(x1, x2, y1, y2)
    gs = g1 + g2

    gs0 = jnp.where(lane_lt8, gs, neg_inf)
    _, v0 = plsc.sort_key_val(gs0, iota, descending=True)
    gs1r = jnp.where(lane_lt8, lax.rev(gs, (0,)), neg_inf)
    _, v1 = plsc.sort_key_val(gs1r, iota, descending=True)
    sel_vals = (v0, v1)

    for tok in range(2):
      tok_base = base + tok * _E
      selk = jnp.where(iota < _KG, sel_vals[tok], jnp.int32(_G))
      sel_asc, _ = plsc.sort_key_val(selk, selk, descending=False)

      cands = []
      for gi in range(_KG):
        g = sel_asc[gi]
        for h in range(2):
          col = g * _EPG + h * 16
          cands.append((buf[pl.ds(tok_base + col, 16)], col))

      lvl = [plsc.sort_key_val(ck, iota, descending=True)[0]
             for ck, _ in cands]
      while len(lvl) > 1:
        lvl = [_merge_vals16(lvl[i], lvl[i + 1], iota)
               for i in range(0, len(lvl), 2)]
      theta = lvl[0][7]

      srow = (pair * 2 + tok) * 32
      stage_k[pl.ds(srow, 16)] = jnp.full((16,), neg_inf, jnp.float32)
      cnt = jnp.int32(0)
      for ck, col in cands:
        m = ck >= theta
        ids = col + iota
        plsc.store_compressed(stage_k.at[pl.ds(srow + cnt, 16)], ck, mask=m)
        plsc.store_compressed(stage_v.at[pl.ds(srow + cnt, 16)], ids, mask=m)
        cnt = jnp.minimum(cnt + plsc.all_reduce_population_count(m)[0],
                          jnp.int32(16))
      fk = stage_k[pl.ds(srow, 16)]
      fv = stage_v[pl.ds(srow, 16)]
      fk, fv = plsc.sort_key_val(fk, fv, descending=True)
      cur_v = jnp.where(lane_lt8, fv, 0)

      s4c_g = plsc.load_gather(buf, [tok_base + cur_v])
      bias_g = plsc.load_gather(bias_v, [cur_v])
      w = s4c_g - bias_g
      wm = jnp.where(lane_lt8, w, jnp.float32(0.0))
      tot = jnp.sum(wm)
      rn = _rcp(jnp.broadcast_to(tot, (16,)) + jnp.float32(1e-20))
      wfin = wm * rn * jnp.float32(_SCALE)
      out_off = pair * 16 + tok * 8
      plsc.store_compressed(w_out.at[pl.ds(out_off, 16)], wfin,
                            mask=lane_lt8)
      plsc.store_compressed(id_out.at[pl.ds(out_off, 16)], cur_v,
                            mask=lane_lt8)

  out_base = wid * (_TPW * _K)
  pltpu.sync_copy(w_out.at[pl.ds(0, _TPW * _K)],
                  w_hbm.at[pl.ds(out_base, _TPW * _K)])
  pltpu.sync_copy(id_out.at[pl.ds(0, _TPW * _K)],
                  id_hbm.at[pl.ds(out_base, _TPW * _K)])


@jax.jit
def _run(logits, bias):
  s4c = pl.pallas_call(
      _tc_body,
      grid=(_T // _TC_BLK,),
      in_specs=[
          pl.BlockSpec((_TC_BLK, _E), lambda i: (i, 0)),
          pl.BlockSpec((1, _E), lambda i: (0, 0)),
      ],
      out_specs=pl.BlockSpec((_TC_BLK * _E // 128, 128), lambda i: (i, 0)),
      out_shape=jax.ShapeDtypeStruct((_T * _E // 128, 128), jnp.float32),
  )(logits, bias.reshape(1, _E))

  mesh = plsc.VectorSubcoreMesh(core_axis_name="c", subcore_axis_name="s")
  kfn = pl.kernel(
      _sc_body,
      out_type=(
          jax.ShapeDtypeStruct((_T * _K,), jnp.float32),
          jax.ShapeDtypeStruct((_T * _K,), jnp.int32),
      ),
      mesh=mesh,
      compiler_params=pltpu.CompilerParams(needs_layout_passes=False),
      scratch_types=[
          pltpu.VMEM((_WORDS,), jnp.float32),
          pltpu.VMEM((_E,), jnp.float32),
          pltpu.VMEM((_TPW * _K + 16,), jnp.float32),
          pltpu.VMEM((_TPW * _K + 16,), jnp.int32),
          pltpu.VMEM((_TPW * 32,), jnp.float32),
          pltpu.VMEM((_TPW * 32,), jnp.int32),
      ],
  )
  return kfn(s4c.reshape(-1), bias)


def kernel(router_logits, e_score_correction_bias):
  w, ids = _run(router_logits.astype(jnp.float32),
                e_score_correction_bias.astype(jnp.float32))
  return w.reshape(_T, _K), ids.reshape(_T, _K)


# speedup vs baseline: 1.3484x; 1.3484x over previous
import jax
import jax.numpy as jnp
from jax import lax
from jax.experimental import pallas as pl
from jax.experimental.pallas import tpu as pltpu
from jax.experimental.pallas import tpu_sc as plsc

_T = 8192
_E = 256
_G = 8
_EPG = 32
_K = 8
_KG = 4
_SCALE = 2.5

_NW = 32
_TPW = _T // _NW
_PAIRS = _TPW // 2
_WORDS = _TPW * _E

_TC_BLK = 1024


def _tc_body(x_ref, b_ref, o_ref):
  x = x_ref[...]
  s4c = jax.nn.sigmoid(x) + b_ref[...]
  o_ref[...] = s4c.reshape(_TC_BLK * _E // 128, 128)


def _rcp(d):
  r = plsc.bitcast(jnp.int32(0x7EF311C7) - plsc.bitcast(d, jnp.int32),
                   jnp.float32)
  for _ in range(3):
    r = r * (jnp.float32(2.0) - d * r)
  return r


def _merge_top16(ak, av, bk, bv):
  rk = lax.rev(bk, (0,))
  rv = lax.rev(bv, (0,))
  keep = ak >= rk
  hi_k = jnp.where(keep, ak, rk)
  hi_v = jnp.where(keep, av, rv)
  return plsc.sort_key_val(hi_k, hi_v, descending=True)


def _sc_body(s4c_hbm, bias_hbm, w_hbm, id_hbm, buf, bias_v, w_out, id_out):
  c = lax.axis_index("c")
  s = lax.axis_index("s")
  wid = s * 2 + c

  pltpu.sync_copy(bias_hbm, bias_v)
  pltpu.sync_copy(s4c_hbm.at[pl.ds(wid * _WORDS, _WORDS)], buf)

  iota = lax.iota(jnp.int32, 16)
  lane_lt8 = iota < 8
  trans_base = jnp.where(lane_lt8, iota * _EPG,
                         _E + (15 - iota) * _EPG)
  neg_inf = jnp.float32(-jnp.inf)

  @plsc.parallel_loop(0, _PAIRS, unroll=2)
  def pair_body(pair):
    base = pl.multiple_of(pair * (2 * _E), 2 * _E)

    gidx0 = trans_base + base
    m1s, m2s = [], []
    for chunk in range(4):
      m1 = jnp.full((16,), neg_inf, jnp.float32)
      m2 = jnp.full((16,), neg_inf, jnp.float32)
      for j in range(chunk * 8, chunk * 8 + 8):
        x = plsc.load_gather(buf, [gidx0 + j])
        m2 = jnp.maximum(m2, jnp.minimum(m1, x))
        m1 = jnp.maximum(m1, x)
      m1s.append(m1)
      m2s.append(m2)

    def comb(a1, a2, b1, b2):
      return (jnp.maximum(a1, b1),
              jnp.maximum(jnp.minimum(a1, b1), jnp.maximum(a2, b2)))

    x1, x2 = comb(m1s[0], m2s[0], m1s[1], m2s[1])
    y1, y2 = comb(m1s[2], m2s[2], m1s[3], m2s[3])
    g1, g2 = comb(x1, x2, y1, y2)
    gs = g1 + g2

    gs0 = jnp.where(lane_lt8, gs, neg_inf)
    _, v0 = plsc.sort_key_val(gs0, iota, descending=True)
    gs1r = jnp.where(lane_lt8, lax.rev(gs, (0,)), neg_inf)
    _, v1 = plsc.sort_key_val(gs1r, iota, descending=True)
    sel_vals = (v0, v1)

    for tok in range(2):
      tok_base = base + tok * _E
      lvl = []
      for gi in range(_KG):
        g = sel_vals[tok][gi]
        for h in range(2):
          col = g * _EPG + h * 16
          ck = buf[pl.ds(tok_base + col, 16)]
          lvl.append(plsc.sort_key_val(ck, col + iota, descending=True))
      while len(lvl) > 1:
        lvl = [_merge_top16(lvl[i][0], lvl[i][1], lvl[i + 1][0],
                            lvl[i + 1][1]) for i in range(0, len(lvl), 2)]
      fk, fv = lvl[0]
      fv, fk = plsc.sort_key_val(fv, fk, descending=False)
      fk, fv = plsc.sort_key_val(fk, fv, descending=True)
      cur_v = jnp.where(lane_lt8, fv, 0)

      s4c_g = plsc.load_gather(buf, [tok_base + cur_v])
      bias_g = plsc.load_gather(bias_v, [cur_v])
      w = s4c_g - bias_g
      wm = jnp.where(lane_lt8, w, jnp.float32(0.0))
      tot = jnp.sum(wm)
      rn = _rcp(jnp.broadcast_to(tot, (16,)) + jnp.float32(1e-20))
      wfin = wm * rn * jnp.float32(_SCALE)
      out_off = pair * 16 + tok * 8
      plsc.store_compressed(w_out.at[pl.ds(out_off, 16)], wfin,
                            mask=lane_lt8)
      plsc.store_compressed(id_out.at[pl.ds(out_off, 16)], cur_v,
                            mask=lane_lt8)

  out_base = wid * (_TPW * _K)
  pltpu.sync_copy(w_out.at[pl.ds(0, _TPW * _K)],
                  w_hbm.at[pl.ds(out_base, _TPW * _K)])
  pltpu.sync_copy(id_out.at[pl.ds(0, _TPW * _K)],
                  id_hbm.at[pl.ds(out_base, _TPW * _K)])


@jax.jit
def _run(logits, bias):
  s4c = pl.pallas_call(
      _tc_body,
      grid=(_T // _TC_BLK,),
      in_specs=[
          pl.BlockSpec((_TC_BLK, _E), lambda i: (i, 0)),
          pl.BlockSpec((1, _E), lambda i: (0, 0)),
      ],
      out_specs=pl.BlockSpec((_TC_BLK * _E // 128, 128), lambda i: (i, 0)),
      out_shape=jax.ShapeDtypeStruct((_T * _E // 128, 128), jnp.float32),
  )(logits, bias.reshape(1, _E))

  mesh = plsc.VectorSubcoreMesh(core_axis_name="c", subcore_axis_name="s")
  kfn = pl.kernel(
      _sc_body,
      out_type=(
          jax.ShapeDtypeStruct((_T * _K,), jnp.float32),
          jax.ShapeDtypeStruct((_T * _K,), jnp.int32),
      ),
      mesh=mesh,
      compiler_params=pltpu.CompilerParams(needs_layout_passes=False),
      scratch_types=[
          pltpu.VMEM((_WORDS,), jnp.float32),
          pltpu.VMEM((_E,), jnp.float32),
          pltpu.VMEM((_TPW * _K + 16,), jnp.float32),
          pltpu.VMEM((_TPW * _K + 16,), jnp.int32),
      ],
  )
  return kfn(s4c.reshape(-1), bias)


def kernel(router_logits, e_score_correction_bias):
  w, ids = _run(router_logits.astype(jnp.float32),
                e_score_correction_bias.astype(jnp.float32))
  return w.reshape(_T, _K), ids.reshape(_T, _K)
